# baseline (device time: 107371 ns/iter reference)
import jax
import jax.numpy as jnp
from jax import lax
from jax.experimental import pallas as pl
from jax.experimental.pallas import tpu as pltpu

N_DEV = 4
N_EXP = 16
N_LOCAL = N_EXP // N_DEV
N_TOK = 2048
D = 512
H = 1024
CHUNK = N_TOK // N_DEV
N_HOPS = 2 * (N_DEV - 1)


def kernel(x, router_W, route_idx, expert_W, shared_W):
    def body(x_ref, rw_ref, idx_ref, ew_ref, sw_ref, out_ref,
             comm_ref, sendbuf_ref, send_sems, recv_sems):
        my_i = lax.axis_index("i")
        left = lax.rem(my_i + N_DEV - 1, N_DEV)
        right = lax.rem(my_i + 1, N_DEV)

        barrier_sem = pltpu.get_barrier_semaphore()
        for nbr in (left, right):
            pl.semaphore_signal(
                barrier_sem, inc=1,
                device_id=(nbr,), device_id_type=pl.DeviceIdType.MESH,
            )
        pl.semaphore_wait(barrier_sem, 2)

        xf = x_ref[:, :]
        scores = jnp.dot(xf, rw_ref[:, :], preferred_element_type=jnp.float32)
        s_max = jnp.max(scores, axis=1, keepdims=True)
        p = jnp.exp(scores - s_max)
        probs = p / jnp.sum(p, axis=1, keepdims=True)

        eid = lax.broadcasted_iota(jnp.int32, (N_TOK, N_EXP), 1)
        routed = idx_ref[:, :]

        xb = (xf * 0.25).astype(jnp.bfloat16)
        acc = jnp.dot(xb, sw_ref[:, :].astype(jnp.bfloat16),
                      preferred_element_type=jnp.float32)
        for j in range(N_LOCAL):
            e = my_i * N_LOCAL + j
            sel = jnp.sum(jnp.where(eid == e, probs, 0.0), axis=1,
                          keepdims=True)
            gate = jnp.where(routed == e, sel, 0.0)
            xs = (xf * gate).astype(jnp.bfloat16)
            acc = acc + jnp.dot(xs, ew_ref[j].astype(jnp.bfloat16),
                                preferred_element_type=jnp.float32)
        out_ref[:, :] = acc

        for h in range(N_DEV - 1):
            sc = lax.rem(my_i - h + 2 * N_DEV, N_DEV)
            rc = lax.rem(my_i - h - 1 + 2 * N_DEV, N_DEV)
            sendbuf_ref[:, :] = out_ref[pl.ds(sc * CHUNK, CHUNK), :].astype(
                jnp.bfloat16)
            rdma = pltpu.make_async_remote_copy(
                src_ref=sendbuf_ref,
                dst_ref=comm_ref.at[h],
                send_sem=send_sems.at[h],
                recv_sem=recv_sems.at[h],
                device_id=(right,),
                device_id_type=pl.DeviceIdType.MESH,
            )
            rdma.start()
            rdma.wait()
            out_ref[pl.ds(rc * CHUNK, CHUNK), :] = (
                out_ref[pl.ds(rc * CHUNK, CHUNK), :]
                + comm_ref[h].astype(jnp.float32))

        own = lax.rem(my_i + 1, N_DEV)
        for h in range(N_DEV - 1):
            hh = (N_DEV - 1) + h
            if h == 0:
                sendbuf_ref[:, :] = out_ref[
                    pl.ds(own * CHUNK, CHUNK), :].astype(jnp.bfloat16)
                src = sendbuf_ref
            else:
                src = comm_ref.at[hh - 1]
            rdma = pltpu.make_async_remote_copy(
                src_ref=src,
                dst_ref=comm_ref.at[hh],
                send_sem=send_sems.at[hh],
                recv_sem=recv_sems.at[hh],
                device_id=(right,),
                device_id_type=pl.DeviceIdType.MESH,
            )
            rdma.start()
            rdma.wait()
            rc = lax.rem(my_i - h + 2 * N_DEV, N_DEV)
            out_ref[pl.ds(rc * CHUNK, CHUNK), :] = comm_ref[hh].astype(
                jnp.float32)

    return pl.pallas_call(
        body,
        out_shape=jax.ShapeDtypeStruct((N_TOK, H), jnp.float32),
        in_specs=[pl.BlockSpec(memory_space=pltpu.VMEM)] * 5,
        out_specs=pl.BlockSpec(memory_space=pltpu.VMEM),
        scratch_shapes=[
            pltpu.VMEM((N_HOPS, CHUNK, H), jnp.bfloat16),
            pltpu.VMEM((CHUNK, H), jnp.bfloat16),
            pltpu.SemaphoreType.DMA((N_HOPS,)),
            pltpu.SemaphoreType.DMA((N_HOPS,)),
        ],
        compiler_params=pltpu.CompilerParams(collective_id=0),
    )(x, router_W, route_idx, expert_W, shared_W)


# device time: 66549 ns/iter; 1.6134x vs baseline; 1.6134x over previous
import jax
import jax.numpy as jnp
from jax import lax
from jax.experimental import pallas as pl
from jax.experimental.pallas import tpu as pltpu

N_DEV = 4
N_EXP = 16
N_LOCAL = N_EXP // N_DEV
N_TOK = 2048
D = 512
H = 1024
CHUNK = N_TOK // N_DEV
HALF = H // 2
N_HOPS = 2 * (N_DEV - 1)


def kernel(x, router_W, route_idx, expert_W, shared_W):
    def body(x_ref, rw_ref, idx_ref, ew_ref, sw_ref, out_ref,
             gate_ref, comm_ref, send_sems, recv_sems):
        my_i = lax.axis_index("i")
        left = lax.rem(my_i + N_DEV - 1, N_DEV)
        right = lax.rem(my_i + 1, N_DEV)

        def cidx(k):
            return lax.rem(my_i + k + 2 * N_DEV, N_DEV)

        barrier_sem = pltpu.get_barrier_semaphore()
        for nbr in (left, right):
            pl.semaphore_signal(
                barrier_sem, inc=1,
                device_id=(nbr,), device_id_type=pl.DeviceIdType.MESH,
            )
        pl.semaphore_wait(barrier_sem, 2)

        scores = jnp.dot(x_ref[:, :], rw_ref[:, :],
                         preferred_element_type=jnp.float32)
        s_max = jnp.max(scores, axis=1, keepdims=True)
        p = jnp.exp(scores - s_max)
        probs = p / jnp.sum(p, axis=1, keepdims=True)
        eid = lax.broadcasted_iota(jnp.int32, (N_TOK, N_EXP), 1)
        gated = jnp.where(eid == idx_ref[:, :], probs, 0.0)
        for j in range(N_LOCAL):
            e = my_i * N_LOCAL + j
            gate_ref[:, j:j + 1] = jnp.sum(
                jnp.where(eid == e, gated, 0.0), axis=1, keepdims=True)

        swb = sw_ref[:, :].astype(jnp.bfloat16)
        ewb = [ew_ref[j].astype(jnp.bfloat16) for j in range(N_LOCAL)]

        def compute_chunk(c):
            rows = pl.ds(c * CHUNK, CHUNK)
            xr = x_ref[rows, :]
            g = gate_ref[rows, :]
            acc = jnp.dot((xr * 0.25).astype(jnp.bfloat16), swb,
                          preferred_element_type=jnp.float32)
            for j in range(N_LOCAL):
                xs = (xr * g[:, j:j + 1]).astype(jnp.bfloat16)
                acc = acc + jnp.dot(xs, ewb[j],
                                    preferred_element_type=jnp.float32)
            out_ref[rows, :] = acc.astype(jnp.bfloat16)

        def start_hop(h, src_r, src_l):
            k = 2 * h
            rdma_r = pltpu.make_async_remote_copy(
                src_ref=src_r, dst_ref=comm_ref.at[k],
                send_sem=send_sems.at[k], recv_sem=recv_sems.at[k],
                device_id=(right,), device_id_type=pl.DeviceIdType.MESH,
            )
            rdma_l = pltpu.make_async_remote_copy(
                src_ref=src_l, dst_ref=comm_ref.at[k + 1],
                send_sem=send_sems.at[k + 1], recv_sem=recv_sems.at[k + 1],
                device_id=(left,), device_id_type=pl.DeviceIdType.MESH,
            )
            rdma_r.start()
            rdma_l.start()
            return rdma_r, rdma_l

        def out_l(c):
            return out_ref.at[pl.ds(c * CHUNK, CHUNK), pl.ds(0, HALF)]

        def out_r(c):
            return out_ref.at[pl.ds(c * CHUNK, CHUNK), pl.ds(HALF, HALF)]

        compute_chunk(cidx(0))
        rs0 = start_hop(0, out_l(cidx(0)), out_r(cidx(0)))
        compute_chunk(cidx(-1))
        compute_chunk(cidx(1))
        rs0[0].wait()
        rs0[1].wait()
        out_l(cidx(-1))[:, :] = out_l(cidx(-1))[:, :] + comm_ref[0]
        out_r(cidx(1))[:, :] = out_r(cidx(1))[:, :] + comm_ref[1]

        rs1 = start_hop(1, out_l(cidx(-1)), out_r(cidx(1)))
        compute_chunk(cidx(2))
        rs1[0].wait()
        rs1[1].wait()
        out_l(cidx(-2))[:, :] = out_l(cidx(-2))[:, :] + comm_ref[2]
        out_r(cidx(2))[:, :] = out_r(cidx(2))[:, :] + comm_ref[3]

        rs2 = start_hop(2, out_l(cidx(-2)), out_r(cidx(2)))
        rs2[0].wait()
        rs2[1].wait()
        out_l(cidx(1))[:, :] = out_l(cidx(1))[:, :] + comm_ref[4]
        out_r(cidx(-1))[:, :] = out_r(cidx(-1))[:, :] + comm_ref[5]

        ag0 = start_hop(3, out_l(cidx(1)), out_r(cidx(-1)))
        ag0[0].wait_recv()
        ag0[1].wait_recv()
        ag1 = start_hop(4, comm_ref.at[6], comm_ref.at[7])
        out_l(cidx(0))[:, :] = comm_ref[6]
        out_r(cidx(0))[:, :] = comm_ref[7]
        ag1[0].wait_recv()
        ag1[1].wait_recv()
        ag2 = start_hop(5, comm_ref.at[8], comm_ref.at[9])
        out_l(cidx(-1))[:, :] = comm_ref[8]
        out_r(cidx(1))[:, :] = comm_ref[9]
        ag2[0].wait_recv()
        ag2[1].wait_recv()
        out_l(cidx(-2))[:, :] = comm_ref[10]
        out_r(cidx(2))[:, :] = comm_ref[11]

        for r in (*ag0, *ag1, *ag2):
            r.wait_send()

    return pl.pallas_call(
        body,
        out_shape=jax.ShapeDtypeStruct((N_TOK, H), jnp.bfloat16),
        in_specs=[pl.BlockSpec(memory_space=pltpu.VMEM)] * 5,
        out_specs=pl.BlockSpec(memory_space=pltpu.VMEM),
        scratch_shapes=[
            pltpu.VMEM((N_TOK, N_LOCAL), jnp.float32),
            pltpu.VMEM((2 * N_HOPS, CHUNK, HALF), jnp.bfloat16),
            pltpu.SemaphoreType.DMA((2 * N_HOPS,)),
            pltpu.SemaphoreType.DMA((2 * N_HOPS,)),
        ],
        compiler_params=pltpu.CompilerParams(collective_id=0),
    )(x, router_W, route_idx, expert_W, shared_W)


# device time: 63403 ns/iter; 1.6935x vs baseline; 1.0496x over previous
import jax
import jax.numpy as jnp
from jax import lax
from jax.experimental import pallas as pl
from jax.experimental.pallas import tpu as pltpu

N_DEV = 4
N_EXP = 16
N_LOCAL = N_EXP // N_DEV
N_TOK = 2048
D = 512
H = 1024
CHUNK = N_TOK // N_DEV
HALF = H // 2
SUB = 2
SUBW = HALF // SUB
N_HOPS = 2 * (N_DEV - 1)
N_SLOTS = N_HOPS * 2 * SUB


def kernel(x, router_W, route_idx, expert_W, shared_W):
    def body(x_ref, rw_ref, idx_ref, ew_ref, sw_ref, out_ref,
             gate_ref, comm_ref, send_sems, recv_sems):
        my_i = lax.axis_index("i")
        left = lax.rem(my_i + N_DEV - 1, N_DEV)
        right = lax.rem(my_i + 1, N_DEV)

        def cidx(k):
            return lax.rem(my_i + k + 2 * N_DEV, N_DEV)

        barrier_sem = pltpu.get_barrier_semaphore()
        for nbr in (left, right):
            pl.semaphore_signal(
                barrier_sem, inc=1,
                device_id=(nbr,), device_id_type=pl.DeviceIdType.MESH,
            )
        pl.semaphore_wait(barrier_sem, 2)

        scores = jnp.dot(x_ref[:, :], rw_ref[:, :],
                         preferred_element_type=jnp.float32)
        s_max = jnp.max(scores, axis=1, keepdims=True)
        p = jnp.exp(scores - s_max)
        probs = p / jnp.sum(p, axis=1, keepdims=True)
        eid = lax.broadcasted_iota(jnp.int32, (N_TOK, N_EXP), 1)
        gated = jnp.where(eid == idx_ref[:, :], probs, 0.0)
        for j in range(N_LOCAL):
            e = my_i * N_LOCAL + j
            gate_ref[:, j:j + 1] = jnp.sum(
                jnp.where(eid == e, gated, 0.0), axis=1, keepdims=True)

        swb = sw_ref[:, :].astype(jnp.bfloat16)
        ewb = [ew_ref[j].astype(jnp.bfloat16) for j in range(N_LOCAL)]

        def compute_chunk(c):
            rows = pl.ds(c * CHUNK, CHUNK)
            xr = x_ref[rows, :]
            g = gate_ref[rows, :]
            acc = jnp.dot((xr * 0.25).astype(jnp.bfloat16), swb,
                          preferred_element_type=jnp.float32)
            for j in range(N_LOCAL):
                xs = (xr * g[:, j:j + 1]).astype(jnp.bfloat16)
                acc = acc + jnp.dot(xs, ewb[j],
                                    preferred_element_type=jnp.float32)
            out_ref[rows, :] = acc.astype(jnp.bfloat16)

        def out_sub(c, d, s):
            return out_ref.at[pl.ds(c * CHUNK, CHUNK),
                              pl.ds(d * HALF + s * SUBW, SUBW)]

        def slot(h, d, s):
            return (h * 2 + d) * SUB + s

        def start(h, d, s, src):
            k = slot(h, d, s)
            rdma = pltpu.make_async_remote_copy(
                src_ref=src, dst_ref=comm_ref.at[k],
                send_sem=send_sems.at[k], recv_sem=recv_sems.at[k],
                device_id=(right if d == 0 else left,),
                device_id_type=pl.DeviceIdType.MESH,
            )
            rdma.start()
            return rdma

        pending = []

        def start_hop(h, c_r, c_l, from_out=True, fwd_from=None):
            rs = []
            for d, c in ((0, c_r), (1, c_l)):
                for s in range(SUB):
                    src = out_sub(c, d, s) if from_out else \
                        comm_ref.at[slot(fwd_from, d, s)]
                    rs.append(start(h, d, s, src))
            pending.extend(rs)
            return rs

        def rs_accum(h, c_r, c_l, rdmas):
            for i, (d, c) in enumerate(((0, c_r), (1, c_l))):
                for s in range(SUB):
                    rdmas[i * SUB + s].wait_recv()
                    dst = out_sub(c, d, s)
                    dst[:, :] = dst[:, :] + comm_ref[slot(h, d, s)]

        compute_chunk(cidx(0))
        rs0 = start_hop(0, cidx(0), cidx(0))
        compute_chunk(cidx(-1))
        compute_chunk(cidx(1))
        rs_accum(0, cidx(-1), cidx(1), rs0)

        rs1 = start_hop(1, cidx(-1), cidx(1))
        compute_chunk(cidx(2))
        rs_accum(1, cidx(-2), cidx(2), rs1)

        rs2 = start_hop(2, cidx(-2), cidx(2))
        ag0 = [None] * 4
        for i, (d, c) in enumerate(((0, cidx(1)), (1, cidx(-1)))):
            for s in range(SUB):
                rs2[i * SUB + s].wait_recv()
                dst = out_sub(c, d, s)
                dst[:, :] = dst[:, :] + comm_ref[slot(2, d, s)]
                ag0[i * SUB + s] = start(3, d, s, dst)
        pending.extend(ag0)

        for h, (off_r, off_l) in ((3, (0, 0)), (4, (-1, 1)), (5, (-2, 2))):
            for i, (d, off) in enumerate(((0, off_r), (1, off_l))):
                for s in range(SUB):
                    k = slot(h, d, s)
                    rdma = (ag0[i * SUB + s] if h == 3 else None)
                    if rdma is not None:
                        rdma.wait_recv()
                    else:
                        pltpu.make_async_remote_copy(
                            src_ref=comm_ref.at[k], dst_ref=comm_ref.at[k],
                            send_sem=send_sems.at[k], recv_sem=recv_sems.at[k],
                            device_id=(right if d == 0 else left,),
                            device_id_type=pl.DeviceIdType.MESH,
                        ).wait_recv()
                    if h < 5:
                        pending.append(
                            start(h + 1, d, s, comm_ref.at[k]))
                    out_sub(cidx(off), d, s)[:, :] = comm_ref[k]

        for r in pending:
            r.wait_send()

    return pl.pallas_call(
        body,
        out_shape=jax.ShapeDtypeStruct((N_TOK, H), jnp.bfloat16),
        in_specs=[pl.BlockSpec(memory_space=pltpu.VMEM)] * 5,
        out_specs=pl.BlockSpec(memory_space=pltpu.VMEM),
        scratch_shapes=[
            pltpu.VMEM((N_TOK, N_LOCAL), jnp.float32),
            pltpu.VMEM((N_SLOTS, CHUNK, SUBW), jnp.bfloat16),
            pltpu.SemaphoreType.DMA((N_SLOTS,)),
            pltpu.SemaphoreType.DMA((N_SLOTS,)),
        ],
        compiler_params=pltpu.CompilerParams(collective_id=0),
    )(x, router_W, route_idx, expert_W, shared_W)


# device time: 59112 ns/iter; 1.8164x vs baseline; 1.0726x over previous
import jax
import jax.numpy as jnp
from jax import lax
from jax.experimental import pallas as pl
from jax.experimental.pallas import tpu as pltpu

N_DEV = 4
N_EXP = 16
N_LOCAL = N_EXP // N_DEV
N_TOK = 2048
D = 512
H = 1024
CHUNK = N_TOK // N_DEV
HALF = H // 2
SUB = 4
SUBW = HALF // SUB
N_HOPS = 2 * (N_DEV - 1)
N_SLOTS = N_HOPS * 2 * SUB


def kernel(x, router_W, route_idx, expert_W, shared_W):
    def body(x_ref, rw_ref, idx_ref, ew_ref, sw_ref, out_ref,
             gate_ref, comm_ref, send_sems, recv_sems):
        my_i = lax.axis_index("i")
        left = lax.rem(my_i + N_DEV - 1, N_DEV)
        right = lax.rem(my_i + 1, N_DEV)

        def cidx(k):
            return lax.rem(my_i + k + 2 * N_DEV, N_DEV)

        barrier_sem = pltpu.get_barrier_semaphore()
        for nbr in (left, right):
            pl.semaphore_signal(
                barrier_sem, inc=1,
                device_id=(nbr,), device_id_type=pl.DeviceIdType.MESH,
            )
        pl.semaphore_wait(barrier_sem, 2)

        scores = jnp.dot(x_ref[:, :], rw_ref[:, :],
                         preferred_element_type=jnp.float32)
        s_max = jnp.max(scores, axis=1, keepdims=True)
        p = jnp.exp(scores - s_max)
        probs = p / jnp.sum(p, axis=1, keepdims=True)
        eid = lax.broadcasted_iota(jnp.int32, (N_TOK, N_EXP), 1)
        gated = jnp.where(eid == idx_ref[:, :], probs, 0.0)
        for j in range(N_LOCAL):
            e = my_i * N_LOCAL + j
            gate_ref[:, j:j + 1] = jnp.sum(
                jnp.where(eid == e, gated, 0.0), axis=1, keepdims=True)

        w_flat = jnp.concatenate(
            [ew_ref[j].astype(jnp.bfloat16) for j in range(N_LOCAL)]
            + [(sw_ref[:, :] * 0.25).astype(jnp.bfloat16)], axis=0)

        def compute_chunk(c):
            rows = pl.ds(c * CHUNK, CHUNK)
            xrb = x_ref[rows, :].astype(jnp.bfloat16)
            gb = gate_ref[rows, :].astype(jnp.bfloat16)
            xg = jnp.concatenate(
                [xrb * gb[:, j:j + 1] for j in range(N_LOCAL)] + [xrb],
                axis=1)
            out_ref[rows, :] = jnp.dot(
                xg, w_flat, preferred_element_type=jnp.float32,
            ).astype(jnp.bfloat16)

        def out_sub(c, d, s):
            return out_ref.at[pl.ds(c * CHUNK, CHUNK),
                              pl.ds(d * HALF + s * SUBW, SUBW)]

        def slot(h, d, s):
            return (h * 2 + d) * SUB + s

        def start(h, d, s, src):
            k = slot(h, d, s)
            rdma = pltpu.make_async_remote_copy(
                src_ref=src, dst_ref=comm_ref.at[k],
                send_sem=send_sems.at[k], recv_sem=recv_sems.at[k],
                device_id=(right if d == 0 else left,),
                device_id_type=pl.DeviceIdType.MESH,
            )
            rdma.start()
            return rdma

        pending = []

        def start_hop(h, c_r, c_l, from_out=True, fwd_from=None):
            rs = []
            for d, c in ((0, c_r), (1, c_l)):
                for s in range(SUB):
                    src = out_sub(c, d, s) if from_out else \
                        comm_ref.at[slot(fwd_from, d, s)]
                    rs.append(start(h, d, s, src))
            pending.extend(rs)
            return rs

        def rs_step(h, c_r, c_l, rdmas):
            nxt = []
            for i, (d, c) in enumerate(((0, c_r), (1, c_l))):
                for s in range(SUB):
                    rdmas[i * SUB + s].wait_recv()
                    dst = out_sub(c, d, s)
                    dst[:, :] = dst[:, :] + comm_ref[slot(h, d, s)]
                    nxt.append(start(h + 1, d, s, dst))
            pending.extend(nxt)
            return nxt

        compute_chunk(cidx(0))
        rs0 = start_hop(0, cidx(0), cidx(0))
        compute_chunk(cidx(-1))
        compute_chunk(cidx(1))
        rs1 = rs_step(0, cidx(-1), cidx(1), rs0)
        compute_chunk(cidx(2))
        rs2 = rs_step(1, cidx(-2), cidx(2), rs1)
        ag0 = rs_step(2, cidx(1), cidx(-1), rs2)

        stores = []
        for h, (off_r, off_l) in ((3, (0, 0)), (4, (-1, 1)), (5, (-2, 2))):
            for i, (d, off) in enumerate(((0, off_r), (1, off_l))):
                for s in range(SUB):
                    k = slot(h, d, s)
                    rdma = (ag0[i * SUB + s] if h == 3 else None)
                    if rdma is not None:
                        rdma.wait_recv()
                    else:
                        pltpu.make_async_remote_copy(
                            src_ref=comm_ref.at[k], dst_ref=comm_ref.at[k],
                            send_sem=send_sems.at[k], recv_sem=recv_sems.at[k],
                            device_id=(right if d == 0 else left,),
                            device_id_type=pl.DeviceIdType.MESH,
                        ).wait_recv()
                    if h < 5:
                        pending.append(
                            start(h + 1, d, s, comm_ref.at[k]))
                    stores.append((cidx(off), d, s, k))
        for c, d, s, k in stores:
            out_sub(c, d, s)[:, :] = comm_ref[k]

        for r in pending:
            r.wait_send()

    return pl.pallas_call(
        body,
        out_shape=jax.ShapeDtypeStruct((N_TOK, H), jnp.bfloat16),
        in_specs=[pl.BlockSpec(memory_space=pltpu.VMEM)] * 5,
        out_specs=pl.BlockSpec(memory_space=pltpu.VMEM),
        scratch_shapes=[
            pltpu.VMEM((N_TOK, N_LOCAL), jnp.float32),
            pltpu.VMEM((N_SLOTS, CHUNK, SUBW), jnp.bfloat16),
            pltpu.SemaphoreType.DMA((N_SLOTS,)),
            pltpu.SemaphoreType.DMA((N_SLOTS,)),
        ],
        compiler_params=pltpu.CompilerParams(collective_id=0),
    )(x, router_W, route_idx, expert_W, shared_W)
